# Initial kernel scaffold; baseline (speedup 1.0000x reference)
import jax
import jax.numpy as jnp
from jax.experimental import pallas as pl


def kernel(point_groups, W1, b1, W2, b2):
    raise NotImplementedError("write your pallas kernel here")


# speedup vs baseline: 1.9784x; 1.9784x over previous
import functools

import jax
import jax.numpy as jnp
from jax.experimental import pallas as pl

B, P, N = 8, 512, 128
IN_DIM, HID, OUT_DIM = 3, 64, 60
G = 16


def _body(x_ref, w1_ref, b1_ref, w2_ref, b2_ref, out_ref):
    x = x_ref[...]
    w1 = w1_ref[...]
    h = b1_ref[...]
    for d in range(IN_DIM):
        h = h + x[:, d:d + 1] * w1[d:d + 1, :]
    h = jnp.maximum(h, 0.0)
    o = jnp.dot(h, w2_ref[...], preferred_element_type=jnp.float32)
    o = o + b2_ref[...]
    o = o.reshape(G, N, OUT_DIM)
    out_ref[...] = jnp.max(o, axis=1)


@functools.partial(jax.jit, static_argnames=())
def kernel(point_groups, W1, b1, W2, b2):
    num_patches = B * P
    x = point_groups.reshape(num_patches * N, IN_DIM)
    grid = (num_patches // G,)
    out = pl.pallas_call(
        _body,
        grid=grid,
        in_specs=[
            pl.BlockSpec((G * N, IN_DIM), lambda i: (i, 0)),
            pl.BlockSpec((IN_DIM, HID), lambda i: (0, 0)),
            pl.BlockSpec((1, HID), lambda i: (0, 0)),
            pl.BlockSpec((HID, OUT_DIM), lambda i: (0, 0)),
            pl.BlockSpec((1, OUT_DIM), lambda i: (0, 0)),
        ],
        out_specs=pl.BlockSpec((G, OUT_DIM), lambda i: (i, 0)),
        out_shape=jax.ShapeDtypeStruct((num_patches, OUT_DIM), jnp.float32),
    )(x, W1, b1.reshape(1, HID), W2, b2.reshape(1, OUT_DIM))
    return out.reshape(B, P, OUT_DIM)


# speedup vs baseline: 2.5319x; 1.2798x over previous
import functools

import jax
import jax.numpy as jnp
from jax.experimental import pallas as pl

B, P, N = 8, 512, 128
IN_DIM, HID, OUT_DIM = 3, 64, 60
G = 64


def _body(x_ref, w1_ref, b1_ref, w2_ref, b2_ref, out_ref):
    x = x_ref[...]
    w1 = w1_ref[...]
    h = jnp.dot(x, w1, preferred_element_type=jnp.float32) + b1_ref[...]
    h = jnp.maximum(h, 0.0)
    o = jnp.dot(h, w2_ref[...], preferred_element_type=jnp.float32)
    o = o + b2_ref[...]
    o = o.reshape(G, N, OUT_DIM)
    out_ref[...] = jnp.max(o, axis=1)


@functools.partial(jax.jit, static_argnames=())
def kernel(point_groups, W1, b1, W2, b2):
    num_patches = B * P
    x = point_groups.reshape(num_patches * N, IN_DIM)
    grid = (num_patches // G,)
    out = pl.pallas_call(
        _body,
        grid=grid,
        in_specs=[
            pl.BlockSpec((G * N, IN_DIM), lambda i: (i, 0)),
            pl.BlockSpec((IN_DIM, HID), lambda i: (0, 0)),
            pl.BlockSpec((1, HID), lambda i: (0, 0)),
            pl.BlockSpec((HID, OUT_DIM), lambda i: (0, 0)),
            pl.BlockSpec((1, OUT_DIM), lambda i: (0, 0)),
        ],
        out_specs=pl.BlockSpec((G, OUT_DIM), lambda i: (i, 0)),
        out_shape=jax.ShapeDtypeStruct((num_patches, OUT_DIM), jnp.float32),
    )(x, W1, b1.reshape(1, HID), W2, b2.reshape(1, OUT_DIM))
    return out.reshape(B, P, OUT_DIM)
